# baseline (speedup 1.0000x reference)
import jax
import jax.numpy as jnp
from jax.experimental import pallas as pl
from jax.experimental.pallas import tpu as pltpu

BETA = 0.25
NUM_TOKENS = 1024
CODE_DIM = 64
BPG = 4


def _vq_kernel(z_ref, emb_ref, idx_ref, zq_ref, loss_ref,
               en2_ref, en_sq_ref):
    g = pl.program_id(0)
    emb = emb_ref[...]

    @pl.when(g == 0)
    def _():
        enorm = jnp.clip(
            jnp.sqrt(jnp.sum(emb * emb, axis=1, keepdims=True)), 1e-12, None)
        en0 = emb / enorm
        en2_ref[...] = -2.0 * en0
        en_sq_ref[...] = jnp.sum(en0 * en0, axis=1, keepdims=True)

    en2 = en2_ref[...]
    en_sq = en_sq_ref[...]

    part = jnp.zeros((1, 1), jnp.float32)
    for i in range(BPG):
        zb = z_ref[i].reshape(CODE_DIM, 1024)

        zsq_raw = jnp.sum(zb * zb, axis=0, keepdims=True)
        znorm = jnp.clip(jnp.sqrt(zsq_raw), 1e-12, None)
        zn = zb / znorm
        zsq = jnp.sum(zn * zn, axis=0, keepdims=True)

        s2 = jax.lax.dot_general(
            en2, zn, (((1,), (0,)), ((), ())),
            preferred_element_type=jnp.float32,
        )
        d = (zsq + en_sq) + s2
        idx = jnp.argmin(d, axis=0)

        onehot = (jax.lax.broadcasted_iota(jnp.int32, (NUM_TOKENS, 1024), 0)
                  == idx[None, :]).astype(jnp.float32)
        zq = jax.lax.dot_general(
            emb, onehot, (((0,), (0,)), ((), ())),
            preferred_element_type=jnp.float32,
        )

        idx_ref[pl.ds(i * 1024, 1024)] = idx
        zq_ref[i] = zq.reshape(CODE_DIM, 32, 32)

        diff = zq - zb
        part = part + jnp.sum(diff * diff).reshape(1, 1)

    loss_ref[...] = jnp.where(g == 0, part, loss_ref[...] + part)


@jax.jit
def kernel(z, embedding_weight):
    B, C, H, W = z.shape
    P = H * W

    idx1, z_q_out, loss_sum = pl.pallas_call(
        _vq_kernel,
        grid=(B // BPG,),
        in_specs=[
            pl.BlockSpec((BPG, C, H, W), lambda g: (g, 0, 0, 0)),
            pl.BlockSpec((NUM_TOKENS, CODE_DIM), lambda g: (0, 0)),
        ],
        out_specs=[
            pl.BlockSpec((BPG * P,), lambda g: (g,)),
            pl.BlockSpec((BPG, C, H, W), lambda g: (g, 0, 0, 0)),
            pl.BlockSpec((1, 1), lambda g: (0, 0)),
        ],
        out_shape=[
            jax.ShapeDtypeStruct((B * P,), jnp.int32),
            jax.ShapeDtypeStruct((B, C, H, W), jnp.float32),
            jax.ShapeDtypeStruct((1, 1), jnp.float32),
        ],
        scratch_shapes=[
            pltpu.VMEM((NUM_TOKENS, CODE_DIM), jnp.float32),
            pltpu.VMEM((NUM_TOKENS, 1), jnp.float32),
        ],
    )(z, embedding_weight)

    m = loss_sum[0, 0] / (B * C * P)
    loss = BETA * m + m
    return (loss, z_q_out, idx1)


# speedup vs baseline: 1.5544x; 1.5544x over previous
import jax
import jax.numpy as jnp
from jax.experimental import pallas as pl
from jax.experimental.pallas import tpu as pltpu

BETA = 0.25
NUM_TOKENS = 1024
CODE_DIM = 64
BPG = 4


def _vq_kernel(z_ref, emb_ref, idx_ref, zq_ref, loss_ref,
               en2_ref, en_sq_ref):
    g = pl.program_id(0)
    emb = emb_ref[...]

    @pl.when(g == 0)
    def _():
        enorm = jnp.clip(
            jnp.sqrt(jnp.sum(emb * emb, axis=1, keepdims=True)), 1e-12, None)
        en0 = emb / enorm
        en2_ref[...] = -2.0 * en0
        en_sq_ref[...] = jnp.sum(en0 * en0, axis=1, keepdims=True)

    en2 = en2_ref[...]
    en_sq = en_sq_ref[...]

    part = jnp.zeros((1, 1), jnp.float32)
    for i in range(BPG):
        zb = z_ref[i]

        zsq_raw = jnp.sum(zb * zb, axis=0, keepdims=True)
        znorm = jnp.clip(jnp.sqrt(zsq_raw), 1e-12, None)
        zn = zb / znorm
        zsq = jnp.sum(zn * zn, axis=0, keepdims=True)

        s2 = jax.lax.dot_general(
            en2, zn, (((1,), (0,)), ((), ())),
            preferred_element_type=jnp.float32,
        )
        d = (zsq + en_sq) + s2
        idx = jnp.argmin(d, axis=0)

        onehot = (jax.lax.broadcasted_iota(jnp.int32, (NUM_TOKENS, 1024), 0)
                  == idx[None, :]).astype(jnp.float32)
        zq = jax.lax.dot_general(
            emb, onehot, (((0,), (0,)), ((), ())),
            preferred_element_type=jnp.float32,
        )

        idx_ref[pl.ds(i * 1024, 1024)] = idx
        zq_ref[i] = zq

        diff = zq - zb
        part = part + jnp.sum(diff * diff).reshape(1, 1)

    loss_ref[...] = jnp.where(g == 0, part, loss_ref[...] + part)


@jax.jit
def kernel(z, embedding_weight):
    B, C, H, W = z.shape
    P = H * W
    z3 = z.reshape(B, C, P)

    idx1, zq3, loss_sum = pl.pallas_call(
        _vq_kernel,
        grid=(B // BPG,),
        in_specs=[
            pl.BlockSpec((BPG, C, P), lambda g: (g, 0, 0)),
            pl.BlockSpec((NUM_TOKENS, CODE_DIM), lambda g: (0, 0)),
        ],
        out_specs=[
            pl.BlockSpec((BPG * P,), lambda g: (g,)),
            pl.BlockSpec((BPG, C, P), lambda g: (g, 0, 0)),
            pl.BlockSpec((1, 1), lambda g: (0, 0)),
        ],
        out_shape=[
            jax.ShapeDtypeStruct((B * P,), jnp.int32),
            jax.ShapeDtypeStruct((B, C, P), jnp.float32),
            jax.ShapeDtypeStruct((1, 1), jnp.float32),
        ],
        scratch_shapes=[
            pltpu.VMEM((NUM_TOKENS, CODE_DIM), jnp.float32),
            pltpu.VMEM((NUM_TOKENS, 1), jnp.float32),
        ],
    )(z3, embedding_weight)

    m = loss_sum[0, 0] / (B * C * P)
    loss = BETA * m + m
    z_q_out = zq3.reshape(B, C, H, W)
    return (loss, z_q_out, idx1)


# speedup vs baseline: 1.5714x; 1.0110x over previous
import jax
import jax.numpy as jnp
from jax.experimental import pallas as pl
from jax.experimental.pallas import tpu as pltpu

BETA = 0.25
NUM_TOKENS = 1024
CODE_DIM = 64
BPG = 8


def _vq_kernel(z_ref, emb_ref, idx_ref, zq_ref, loss_ref,
               en2_ref, en_sq_ref):
    g = pl.program_id(0)
    emb = emb_ref[...]

    @pl.when(g == 0)
    def _():
        enorm = jnp.clip(
            jnp.sqrt(jnp.sum(emb * emb, axis=1, keepdims=True)), 1e-12, None)
        en0 = emb / enorm
        en2_ref[...] = -2.0 * en0
        en_sq_ref[...] = jnp.sum(en0 * en0, axis=1, keepdims=True)

    en2 = en2_ref[...]
    en_sq = en_sq_ref[...]

    part = jnp.zeros((1, 1), jnp.float32)
    for i in range(BPG):
        zb = z_ref[i]

        zsq_raw = jnp.sum(zb * zb, axis=0, keepdims=True)
        znorm = jnp.clip(jnp.sqrt(zsq_raw), 1e-12, None)
        zn = zb / znorm
        zsq = jnp.sum(zn * zn, axis=0, keepdims=True)

        s2 = jax.lax.dot_general(
            en2, zn, (((1,), (0,)), ((), ())),
            preferred_element_type=jnp.float32,
        )
        d = (zsq + en_sq) + s2
        idx = jnp.argmin(d, axis=0)

        onehot = (jax.lax.broadcasted_iota(jnp.int32, (NUM_TOKENS, 1024), 0)
                  == idx[None, :]).astype(jnp.float32)
        zq = jax.lax.dot_general(
            emb, onehot, (((0,), (0,)), ((), ())),
            preferred_element_type=jnp.float32,
        )

        idx_ref[pl.ds(i * 1024, 1024)] = idx
        zq_ref[i] = zq

        diff = zq - zb
        part = part + jnp.sum(diff * diff).reshape(1, 1)

    loss_ref[...] = jnp.where(g == 0, part, loss_ref[...] + part)


@jax.jit
def kernel(z, embedding_weight):
    B, C, H, W = z.shape
    P = H * W
    z3 = z.reshape(B, C, P)

    idx1, zq3, loss_sum = pl.pallas_call(
        _vq_kernel,
        grid=(B // BPG,),
        in_specs=[
            pl.BlockSpec((BPG, C, P), lambda g: (g, 0, 0)),
            pl.BlockSpec((NUM_TOKENS, CODE_DIM), lambda g: (0, 0)),
        ],
        out_specs=[
            pl.BlockSpec((BPG * P,), lambda g: (g,)),
            pl.BlockSpec((BPG, C, P), lambda g: (g, 0, 0)),
            pl.BlockSpec((1, 1), lambda g: (0, 0)),
        ],
        out_shape=[
            jax.ShapeDtypeStruct((B * P,), jnp.int32),
            jax.ShapeDtypeStruct((B, C, P), jnp.float32),
            jax.ShapeDtypeStruct((1, 1), jnp.float32),
        ],
        scratch_shapes=[
            pltpu.VMEM((NUM_TOKENS, CODE_DIM), jnp.float32),
            pltpu.VMEM((NUM_TOKENS, 1), jnp.float32),
        ],
    )(z3, embedding_weight)

    m = loss_sum[0, 0] / (B * C * P)
    loss = BETA * m + m
    z_q_out = zq3.reshape(B, C, H, W)
    return (loss, z_q_out, idx1)
